# baseline (speedup 1.0000x reference)
import jax
import jax.numpy as jnp
from jax import lax
from jax.experimental import pallas as pl

NC = 1000
B0 = 1024
B1 = 26


RB = 2
NS = B1 // RB


def _onehot_t(idxt_ref, out_ref):
    rows = idxt_ref[0]
    iota = lax.broadcasted_iota(jnp.int32, (RB, NC, B0), 1)
    cmp = rows[:, None, :] == iota
    out_ref[...] = cmp.astype(out_ref.dtype)


def kernel(x):
    xt = x.T.reshape(NS, RB, B0)
    out_t = pl.pallas_call(
        _onehot_t,
        grid=(NS,),
        in_specs=[pl.BlockSpec((1, RB, B0), lambda i: (i, 0, 0))],
        out_specs=pl.BlockSpec((RB, NC, B0), lambda i: (i, 0, 0)),
        out_shape=jax.ShapeDtypeStruct((B1, NC, B0), x.dtype),
    )(xt)
    return jnp.transpose(out_t, (2, 0, 1))
